# baseline (speedup 1.0000x reference)
import functools

import jax
import jax.numpy as jnp
from jax import lax
from jax.experimental import pallas as pl
from jax.experimental.pallas import tpu as pltpu
from jax.experimental.pallas import tpu_sc as plsc

NC = 2
NS = 16
NW = NC * NS

CHUNK = 400
N_STREAMS = 10
IPS = CHUNK // N_STREAMS
NBUF = 2
LANES = 16


def _build_sc_call(L, D, out_rows, chunk_out, n_chunks):
    d_regs = D // LANES
    seqs = CHUNK // L
    mesh = plsc.VectorSubcoreMesh(core_axis_name="c", subcore_axis_name="s")

    @functools.partial(
        pl.kernel,
        out_type=jax.ShapeDtypeStruct((out_rows, 128), jnp.float32),
        mesh=mesh,
        scratch_types=(
            [pltpu.VMEM((16, 128), jnp.int32)] * NBUF
            + [pltpu.VMEM((CHUNK, 128), jnp.float32)] * NBUF
            + [
                pltpu.VMEM((64,), jnp.int32),
                pltpu.VMEM((64, 128), jnp.float32),
            ]
            + [pltpu.SemaphoreType.DMA] * (2 * NBUF + 1)
        ),
    )
    def sc_embed(table_hbm, ids_hbm, pos_hbm, m1_hbm, m2_hbm, out_hbm,
                 idx0, idx1, stg0, stg1, pos_v, psum_v,
                 g0, g1, w0, w1, psem):
        idxs = (idx0, idx1)
        stgs = (stg0, stg1)
        gsems = (g0, g1)
        wsems = (w0, w1)

        wid = lax.axis_index("s") * NC + lax.axis_index("c")

        pltpu.sync_copy(pos_hbm, pos_v)
        cp1 = pltpu.async_copy(m1_hbm.at[pos_v], psum_v, psem)
        cp2 = pltpu.async_copy(m2_hbm.at[pos_v], stg0.at[pl.ds(0, 64)], psem)
        cp1.wait()
        cp2.wait()

        def possum_body(i, carry):
            for d in range(d_regs):
                sl = pl.ds(d * LANES, LANES)
                psum_v[i, sl] = psum_v[i, sl] + stg0[i, sl]
            return carry

        lax.fori_loop(0, L, possum_body, 0)

        def fire_gather(b, ci):
            pltpu.sync_copy(ids_hbm.at[wid, ci], idxs[b])
            for s in range(N_STREAMS):
                pltpu.async_copy(
                    table_hbm.at[idxs[b].at[s, pl.ds(0, IPS)]],
                    stgs[b].at[pl.ds(s * IPS, IPS)],
                    gsems[b],
                )

        def drain_gather(b):
            pltpu.make_async_copy(
                out_hbm.at[pl.ds(0, CHUNK)], stgs[b], gsems[b]).wait()

        def fire_wb(b, ci):
            base = (wid * n_chunks + ci) * chunk_out
            pltpu.async_copy(stgs[b].at[pl.ds(0, chunk_out)],
                             out_hbm.at[pl.ds(base, chunk_out)], wsems[b])

        def drain_wb(b):
            pltpu.make_async_copy(
                out_hbm.at[pl.ds(0, chunk_out)],
                stgs[b].at[pl.ds(0, chunk_out)], wsems[b]).wait()

        def compute(b):
            for s in range(seqs):
                def add_body(l2, carry, s=s):
                    for lo in range(2):
                        l = 2 * l2 + lo
                        for d in range(d_regs):
                            src = pl.ds(d * LANES, LANES)
                            dst = pl.ds(lo * 64 + d * LANES, LANES)
                            stgs[b][s * 25 + l2, dst] = (
                                stgs[b][s * 50 + l, src] + psum_v[l, src])
                    return carry

                lax.fori_loop(0, L // 2, add_body, 0)

        fire_gather(0, 0)

        def round_body(r, carry):
            for b in range(NBUF):
                ci = r * NBUF + b
                b2 = 1 - b
                drain_gather(b)

                @pl.when(ci + 1 < n_chunks)
                def _prefetch():
                    @pl.when(ci >= 1)
                    def _reclaim():
                        drain_wb(b2)

                    fire_gather(b2, ci + 1)

                compute(b)
                fire_wb(b, ci)

            return carry

        lax.fori_loop(0, n_chunks // NBUF, round_body, 0)
        for b in range(NBUF):
            drain_wb(b)

    return sc_embed


def kernel(input_ids, position_ids, subword_table, m1_table, m2_table):
    B, L = input_ids.shape
    V, D = subword_table.shape
    total_rows = B * L

    rows_per_worker = total_rows // NW
    n_chunks = rows_per_worker // CHUNK
    chunk_out = CHUNK * D // 128
    out_rows = total_rows * D // 128
    assert rows_per_worker * NW == total_rows
    assert n_chunks * CHUNK == rows_per_worker and n_chunks % NBUF == 0
    assert CHUNK % (2 * L) == 0 and CHUNK % 16 == 0 and D == 64

    table_p = jnp.pad(subword_table, ((0, 0), (0, 128 - D)))
    m1p = jnp.pad(m1_table, ((0, (-m1_table.shape[0]) % 8), (0, 128 - D)))
    m2p = jnp.pad(m2_table, ((0, (-m2_table.shape[0]) % 8), (0, 128 - D)))

    ids32 = input_ids.astype(jnp.int32).reshape(NW, n_chunks, N_STREAMS, IPS)
    ids_p = jnp.zeros((NW, n_chunks, 16, 128), jnp.int32).at[
        :, :, :N_STREAMS, :IPS].set(ids32)
    pos_p = jnp.zeros((64,), jnp.int32).at[:L].set(
        position_ids.astype(jnp.int32))

    sc_embed = _build_sc_call(L, D, out_rows, chunk_out, n_chunks)
    out = sc_embed(table_p, ids_p, pos_p, m1p, m2p)
    return out.reshape(B, L, D)


# speedup vs baseline: 1.3382x; 1.3382x over previous
import functools

import jax
import jax.numpy as jnp
from jax import lax
from jax.experimental import pallas as pl
from jax.experimental.pallas import tpu as pltpu
from jax.experimental.pallas import tpu_sc as plsc

NC = 2
NS = 16
NW = NC * NS

SEQS_PER_CHUNK = 8
N_STREAMS = 16
NBUF = 4
LANES = 16


def _build_sc_call(B, L, D, total_rows, chunk_rows, n_chunks, idx_per_stream,
                   pos_pad):
    d_regs = D // LANES
    mesh = plsc.VectorSubcoreMesh(core_axis_name="c", subcore_axis_name="s")

    @functools.partial(
        pl.kernel,
        out_type=jax.ShapeDtypeStruct((total_rows, D), jnp.float32),
        mesh=mesh,
        scratch_types=(
            [pltpu.VMEM((N_STREAMS, idx_per_stream), jnp.int32)] * NBUF
            + [pltpu.VMEM((chunk_rows, D), jnp.float32)] * NBUF
            + [
                pltpu.VMEM((pos_pad,), jnp.int32),
                pltpu.VMEM((pos_pad, D), jnp.float32),
                pltpu.VMEM((pos_pad, D), jnp.float32),
            ]
            + [pltpu.SemaphoreType.DMA] * (2 * NBUF + 1)
        ),
        compiler_params=pltpu.CompilerParams(use_tc_tiling_on_sc=False),
    )
    def sc_embed(table_hbm, ids_hbm, pos_hbm, m1_hbm, m2_hbm, out_hbm,
                 idx0, idx1, idx2, idx3, rows0, rows1, rows2, rows3,
                 pos_v, psum_v, m2r_v,
                 g0, g1, g2, g3, w0, w1, w2, w3, psem):
        idxs = (idx0, idx1, idx2, idx3)
        rows = (rows0, rows1, rows2, rows3)
        gsems = (g0, g1, g2, g3)
        wsems = (w0, w1, w2, w3)

        wid = lax.axis_index("s") * NC + lax.axis_index("c")

        pltpu.sync_copy(pos_hbm, pos_v)
        cp1 = pltpu.async_copy(m1_hbm.at[pos_v], psum_v, psem)
        cp2 = pltpu.async_copy(m2_hbm.at[pos_v], m2r_v, psem)
        cp1.wait()
        cp2.wait()

        def possum_body(i, carry):
            for d in range(d_regs):
                sl = pl.ds(d * LANES, LANES)
                psum_v[i, sl] = psum_v[i, sl] + m2r_v[i, sl]
            return carry

        lax.fori_loop(0, L, possum_body, 0)

        def fire_gather(b, ci):
            pltpu.sync_copy(ids_hbm.at[wid, ci], idxs[b])
            for j in range(N_STREAMS):
                pltpu.async_copy(
                    table_hbm.at[idxs[b].at[j]],
                    rows[b].at[pl.ds(j * idx_per_stream, idx_per_stream)],
                    gsems[b],
                )

        def drain_gather(b):
            pltpu.make_async_copy(
                out_hbm.at[pl.ds(0, chunk_rows)], rows[b], gsems[b]).wait()

        def fire_wb(b, ci):
            base = (wid * n_chunks + ci) * chunk_rows
            pltpu.async_copy(rows[b], out_hbm.at[pl.ds(base, chunk_rows)],
                             wsems[b])

        def drain_wb(b):
            pltpu.make_async_copy(
                out_hbm.at[pl.ds(0, chunk_rows)], rows[b], wsems[b]).wait()

        def add_chunk(b):
            def add_body(l, carry):
                for d in range(d_regs):
                    sl = pl.ds(d * LANES, LANES)
                    p = psum_v[l, sl]
                    for s in range(SEQS_PER_CHUNK):
                        row = s * L + l
                        rows[b][row, sl] = rows[b][row, sl] + p
                return carry

            lax.fori_loop(0, L, add_body, 0)

        fire_gather(0, 0)
        fire_gather(1, 1)

        def round_body(r, carry):
            for b in range(NBUF):
                ci = r * NBUF + b
                drain_gather(b)
                add_chunk(b)
                fire_wb(b, ci)

                b2 = (b + 2) % NBUF

                @pl.when(ci + 2 < n_chunks)
                def _prefetch():
                    @pl.when(ci >= 2)
                    def _reclaim():
                        drain_wb(b2)

                    fire_gather(b2, ci + 2)

            return carry

        lax.fori_loop(0, n_chunks // NBUF, round_body, 0)
        for b in range(NBUF):
            drain_wb(b)

    return sc_embed


def kernel(input_ids, position_ids, subword_table, m1_table, m2_table):
    B, L = input_ids.shape
    D = subword_table.shape[1]
    total_rows = B * L

    rows_per_worker = total_rows // NW
    chunk_rows = SEQS_PER_CHUNK * L
    n_chunks = rows_per_worker // chunk_rows
    idx_per_stream = chunk_rows // N_STREAMS
    assert rows_per_worker * NW == total_rows
    assert n_chunks * chunk_rows == rows_per_worker
    assert n_chunks % NBUF == 0 and n_chunks >= 2 * NBUF
    assert idx_per_stream * N_STREAMS == chunk_rows and idx_per_stream <= 128

    ids = input_ids.astype(jnp.int32).reshape(NW, n_chunks, N_STREAMS,
                                              idx_per_stream)
    pos_pad = 64
    pos = jnp.zeros((pos_pad,), jnp.int32).at[:L].set(
        position_ids.astype(jnp.int32))

    sc_embed = _build_sc_call(B, L, D, total_rows, chunk_rows, n_chunks,
                              idx_per_stream, pos_pad)
    out = sc_embed(subword_table, ids, pos, m1_table, m2_table)
    return out.reshape(B, L, D)


# speedup vs baseline: 1.3510x; 1.0096x over previous
import functools

import jax
import jax.numpy as jnp
from jax import lax
from jax.experimental import pallas as pl
from jax.experimental.pallas import tpu as pltpu
from jax.experimental.pallas import tpu_sc as plsc

NC = 2
NS = 16
NW = NC * NS

SEQS_PER_CHUNK = 8
N_STREAMS = 8
NBUF = 4
LANES = 16


def _build_sc_call(B, L, D, total_rows, chunk_rows, n_chunks, idx_per_stream,
                   pos_pad):
    d_regs = D // LANES
    mesh = plsc.VectorSubcoreMesh(core_axis_name="c", subcore_axis_name="s")

    @functools.partial(
        pl.kernel,
        out_type=jax.ShapeDtypeStruct((total_rows, D), jnp.float32),
        mesh=mesh,
        scratch_types=(
            [pltpu.VMEM((N_STREAMS, idx_per_stream), jnp.int32)] * NBUF
            + [pltpu.VMEM((chunk_rows, D), jnp.float32)] * NBUF
            + [
                pltpu.VMEM((pos_pad,), jnp.int32),
                pltpu.VMEM((pos_pad, D), jnp.float32),
                pltpu.VMEM((pos_pad, D), jnp.float32),
            ]
            + [pltpu.SemaphoreType.DMA] * (2 * NBUF + 1)
        ),
        compiler_params=pltpu.CompilerParams(use_tc_tiling_on_sc=False),
    )
    def sc_embed(table_hbm, ids_hbm, pos_hbm, m1_hbm, m2_hbm, out_hbm,
                 idx0, idx1, idx2, idx3, rows0, rows1, rows2, rows3,
                 pos_v, psum_v, m2r_v,
                 g0, g1, g2, g3, w0, w1, w2, w3, psem):
        idxs = (idx0, idx1, idx2, idx3)
        rows = (rows0, rows1, rows2, rows3)
        gsems = (g0, g1, g2, g3)
        wsems = (w0, w1, w2, w3)

        wid = lax.axis_index("s") * NC + lax.axis_index("c")

        pltpu.sync_copy(pos_hbm, pos_v)
        cp1 = pltpu.async_copy(m1_hbm.at[pos_v], psum_v, psem)
        cp2 = pltpu.async_copy(m2_hbm.at[pos_v], m2r_v, psem)
        cp1.wait()
        cp2.wait()

        def possum_body(i, carry):
            for d in range(d_regs):
                sl = pl.ds(d * LANES, LANES)
                psum_v[i, sl] = psum_v[i, sl] + m2r_v[i, sl]
            return carry

        lax.fori_loop(0, L, possum_body, 0)

        def fire_gather(b, ci):
            pltpu.sync_copy(ids_hbm.at[wid, ci], idxs[b])
            for j in range(N_STREAMS):
                pltpu.async_copy(
                    table_hbm.at[idxs[b].at[j]],
                    rows[b].at[pl.ds(j * idx_per_stream, idx_per_stream)],
                    gsems[b],
                )

        def drain_gather(b):
            pltpu.make_async_copy(
                out_hbm.at[pl.ds(0, chunk_rows)], rows[b], gsems[b]).wait()

        def fire_wb(b, ci):
            base = (wid * n_chunks + ci) * chunk_rows
            pltpu.async_copy(rows[b], out_hbm.at[pl.ds(base, chunk_rows)],
                             wsems[b])

        def drain_wb(b):
            pltpu.make_async_copy(
                out_hbm.at[pl.ds(0, chunk_rows)], rows[b], wsems[b]).wait()

        def add_chunk(b):
            def add_body(l, carry):
                for d in range(d_regs):
                    sl = pl.ds(d * LANES, LANES)
                    p = psum_v[l, sl]
                    for s in range(SEQS_PER_CHUNK):
                        row = s * L + l
                        rows[b][row, sl] = rows[b][row, sl] + p
                return carry

            lax.fori_loop(0, L, add_body, 0)

        fire_gather(0, 0)
        fire_gather(1, 1)

        def round_body(r, carry):
            for b in range(NBUF):
                ci = r * NBUF + b
                drain_gather(b)
                add_chunk(b)
                fire_wb(b, ci)

                b2 = (b + 2) % NBUF

                @pl.when(ci + 2 < n_chunks)
                def _prefetch():
                    @pl.when(ci >= 2)
                    def _reclaim():
                        drain_wb(b2)

                    fire_gather(b2, ci + 2)

            return carry

        lax.fori_loop(0, n_chunks // NBUF, round_body, 0)
        for b in range(NBUF):
            drain_wb(b)

    return sc_embed


def kernel(input_ids, position_ids, subword_table, m1_table, m2_table):
    B, L = input_ids.shape
    D = subword_table.shape[1]
    total_rows = B * L

    rows_per_worker = total_rows // NW
    chunk_rows = SEQS_PER_CHUNK * L
    n_chunks = rows_per_worker // chunk_rows
    idx_per_stream = chunk_rows // N_STREAMS
    assert rows_per_worker * NW == total_rows
    assert n_chunks * chunk_rows == rows_per_worker
    assert n_chunks % NBUF == 0 and n_chunks >= 2 * NBUF
    assert idx_per_stream * N_STREAMS == chunk_rows and idx_per_stream <= 128

    ids = input_ids.astype(jnp.int32).reshape(NW, n_chunks, N_STREAMS,
                                              idx_per_stream)
    pos_pad = 64
    pos = jnp.zeros((pos_pad,), jnp.int32).at[:L].set(
        position_ids.astype(jnp.int32))

    sc_embed = _build_sc_call(B, L, D, total_rows, chunk_rows, n_chunks,
                              idx_per_stream, pos_pad)
    out = sc_embed(subword_table, ids, pos, m1_table, m2_table)
    return out.reshape(B, L, D)
